# baseline (speedup 1.0000x reference)
import functools

import jax
import jax.numpy as jnp
from jax import lax
from jax.experimental import pallas as pl
from jax.experimental.pallas import tpu as pltpu
from jax.experimental.pallas import tpu_sc as plsc

NUM_USERS = 100000
EMBED_DIM = 64
BATCH = 16384

_NC = 2
_NS = 16
_NW = _NC * _NS
_D_PER_W = EMBED_DIM // _NW
_CHUNK = 4096
_N_CHUNKS = BATCH // _CHUNK
_GRP = 16

_mesh = plsc.VectorSubcoreMesh(core_axis_name="c", subcore_axis_name="s")


@functools.partial(
    pl.kernel,
    mesh=_mesh,
    out_type=jax.ShapeDtypeStruct((EMBED_DIM, BATCH), jnp.float32),
    scratch_types=[
        pltpu.VMEM((NUM_USERS,), jnp.float32),
        pltpu.VMEM((BATCH,), jnp.int32),
        pltpu.VMEM((2, _CHUNK), jnp.float32),
        pltpu.VMEM_SHARED((BATCH,), jnp.int32),
        pltpu.SemaphoreType.DMA,
        pltpu.SemaphoreType.DMA,
    ],
    compiler_params=pltpu.CompilerParams(
        needs_layout_passes=False, skip_device_barrier=True
    ),
)
def _gather_kernel(idx_hbm, tab_t_hbm, out_t_hbm, row_v, idx_v, out_v,
                   idx_sh, sem_idx, sem_out):
    wid = lax.axis_index("s") * _NC + lax.axis_index("c")
    sid = lax.axis_index("s")

    @pl.when(sid == 0)
    def _():
        pltpu.sync_copy(idx_hbm, idx_sh)

    out_cps = [None, None]
    for r in range(_D_PER_W):
        d = wid * _D_PER_W + r
        pltpu.sync_copy(tab_t_hbm.at[d], row_v)
        if r == 0:
            plsc.subcore_barrier()
            pltpu.async_copy(idx_sh, idx_v, sem_idx).wait()
        for h in range(_N_CHUNKS):
            buf = h % 2
            if out_cps[buf] is not None:
                out_cps[buf].wait()
                out_cps[buf] = None

            def gather_block(k, carry, h=h, buf=buf):
                base = h * _CHUNK + k * (16 * _GRP)
                vecs = [idx_v[pl.ds(base + 16 * j, 16)] for j in range(_GRP)]
                gs = [plsc.load_gather(row_v, [v]) for v in vecs]
                off = k * (16 * _GRP)
                for j in range(_GRP):
                    out_v[buf, pl.ds(off + 16 * j, 16)] = gs[j]
                return carry

            lax.fori_loop(0, _CHUNK // (16 * _GRP), gather_block, 0)
            out_cps[buf] = pltpu.async_copy(
                out_v.at[buf],
                out_t_hbm.at[d, pl.ds(h * _CHUNK, _CHUNK)],
                sem_out,
            )
    for cp in out_cps:
        if cp is not None:
            cp.wait()


def kernel(idx, embedding_table):
    out_t = _gather_kernel(idx.astype(jnp.int32), embedding_table.T)
    return out_t.T
